# baseline (speedup 1.0000x reference)
import functools

import jax
import jax.numpy as jnp
from jax import lax
from jax.experimental import pallas as pl
from jax.experimental.pallas import tpu as pltpu
from jax.experimental.pallas import tpu_sc as plsc

_HIDDEN = 4096
_N_EXPERTS = 64
_TOP_K = 8
_BLOCK = 1024
_CHUNKS = 4
_LANES = 16


def _matmul_body(x_ref, w_ref, logits_ref):
    logits_ref[...] = lax.dot_general(
        x_ref[...], w_ref[...], (((1,), (1,)), ((), ())),
        preferred_element_type=jnp.float32,
    )


def _tc_logits(x, w):
    n = x.shape[0]
    return pl.pallas_call(
        _matmul_body,
        grid=(n // _BLOCK,),
        in_specs=[
            pl.BlockSpec((_BLOCK, _HIDDEN), lambda i: (i, 0)),
            pl.BlockSpec((_N_EXPERTS, _HIDDEN), lambda i: (0, 0)),
        ],
        out_specs=pl.BlockSpec((_BLOCK, _N_EXPERTS), lambda i: (i, 0)),
        out_shape=jax.ShapeDtypeStruct((n, _N_EXPERTS), jnp.float32),
        compiler_params=pltpu.CompilerParams(
            dimension_semantics=("arbitrary",),
        ),
    )(x, w)


def _merge_desc(ka, va, kb, vb):
    kbr = lax.rev(kb, (0,))
    vbr = lax.rev(vb, (0,))
    take_a = ka >= kbr
    k = jnp.where(take_a, ka, kbr)
    v = jnp.where(take_a, va, vbr)
    return plsc.sort_key_val(k, v, descending=True)


def _sc_route(logits):
    rows = logits.shape[0]
    info = plsc.get_sparse_core_info()
    n_cores, n_subcores = info.num_cores, info.num_subcores
    n_workers = n_cores * n_subcores
    rpw = rows // n_workers
    mesh = plsc.VectorSubcoreMesh(core_axis_name="c", subcore_axis_name="s")

    @functools.partial(
        pl.kernel,
        out_type=(
            jax.ShapeDtypeStruct((rows * _TOP_K,), jnp.float32),
            jax.ShapeDtypeStruct((rows * _TOP_K,), jnp.int32),
        ),
        mesh=mesh,
        scratch_types=[
            pltpu.VMEM((rpw, _N_EXPERTS), jnp.float32),
            pltpu.VMEM((rpw * _TOP_K + 2 * _LANES,), jnp.float32),
            pltpu.VMEM((rpw * _TOP_K + 2 * _LANES,), jnp.int32),
        ],
        compiler_params=pltpu.CompilerParams(needs_layout_passes=False),
    )
    def route(lg_hbm, tv_hbm, ti_hbm, lg_v, tv_v, ti_v):
        wid = lax.axis_index("s") * n_cores + lax.axis_index("c")
        base = wid * rpw
        pltpu.sync_copy(lg_hbm.at[pl.ds(base, rpw)], lg_v)
        lane = lax.iota(jnp.int32, _LANES)
        top_mask = lane < _TOP_K

        def body(r, carry):
            lg = [lg_v[r, pl.ds(c * _LANES, _LANES)] for c in range(4)]
            m = jnp.max(
                jnp.maximum(jnp.maximum(lg[0], lg[1]), jnp.maximum(lg[2], lg[3]))
            )
            e = [jnp.exp(l - m) for l in lg]
            s = jnp.sum((e[0] + e[1]) + (e[2] + e[3]))
            ks, vs = [], []
            for c in range(4):
                p = e[c] / s
                idx = lane + c * _LANES
                key = jnp.where(p > 0, plsc.bitcast(p, jnp.int32), -idx)
                kk, vv = plsc.sort_key_val(key, idx, descending=True)
                ks.append(kk)
                vs.append(vv)
            k01, v01 = _merge_desc(ks[0], vs[0], ks[1], vs[1])
            k23, v23 = _merge_desc(ks[2], vs[2], ks[3], vs[3])
            kf, vf = _merge_desc(k01, v01, k23, v23)
            tv = jnp.where(kf > 0, plsc.bitcast(kf, jnp.float32), 0.0)
            tv = jnp.where(top_mask, tv, 0.0)
            tv = tv / jnp.sum(tv)
            plsc.store_compressed(
                tv_v.at[pl.ds(r * _TOP_K, _LANES)], tv, mask=top_mask
            )
            plsc.store_compressed(
                ti_v.at[pl.ds(r * _TOP_K, _LANES)], vf, mask=top_mask
            )
            return carry

        lax.fori_loop(0, rpw, body, 0, unroll=4)
        pltpu.sync_copy(
            tv_v.at[pl.ds(0, rpw * _TOP_K)],
            tv_hbm.at[pl.ds(base * _TOP_K, rpw * _TOP_K)],
        )
        pltpu.sync_copy(
            ti_v.at[pl.ds(0, rpw * _TOP_K)],
            ti_hbm.at[pl.ds(base * _TOP_K, rpw * _TOP_K)],
        )

    return route(logits)


def kernel(hidden_states, weight):
    x = hidden_states.reshape(-1, _HIDDEN)
    n = x.shape[0]
    chunk = n // _CHUNKS
    lg_parts, tv_parts, ti_parts = [], [], []
    for i in range(_CHUNKS):
        lg = _tc_logits(lax.slice(x, (i * chunk, 0), ((i + 1) * chunk, _HIDDEN)), weight)
        tv_flat, ti_flat = _sc_route(lg)
        lg_parts.append(lg)
        tv_parts.append(tv_flat.reshape(chunk, _TOP_K))
        ti_parts.append(ti_flat.reshape(chunk, _TOP_K))
    if _CHUNKS == 1:
        return lg_parts[0], tv_parts[0], ti_parts[0]
    return (
        jnp.concatenate(lg_parts, 0),
        jnp.concatenate(tv_parts, 0),
        jnp.concatenate(ti_parts, 0),
    )


# speedup vs baseline: 1.7816x; 1.7816x over previous
import functools

import jax
import jax.numpy as jnp
from jax import lax
from jax.experimental import pallas as pl
from jax.experimental.pallas import tpu as pltpu
from jax.experimental.pallas import tpu_sc as plsc

_HIDDEN = 4096
_N_EXPERTS = 64
_TOP_K = 8
_BLOCK = 1024
_CHUNKS = 1
_LANES = 16


def _matmul_body(x_ref, w_ref, logits_ref):
    logits_ref[...] = lax.dot_general(
        x_ref[...], w_ref[...], (((1,), (1,)), ((), ())),
        preferred_element_type=jnp.float32,
    )


def _tc_logits(x, w):
    n = x.shape[0]
    return pl.pallas_call(
        _matmul_body,
        grid=(n // _BLOCK,),
        in_specs=[
            pl.BlockSpec((_BLOCK, _HIDDEN), lambda i: (i, 0)),
            pl.BlockSpec((_N_EXPERTS, _HIDDEN), lambda i: (0, 0)),
        ],
        out_specs=pl.BlockSpec((_BLOCK, _N_EXPERTS), lambda i: (i, 0)),
        out_shape=jax.ShapeDtypeStruct((n, _N_EXPERTS), jnp.float32),
        compiler_params=pltpu.CompilerParams(
            dimension_semantics=("arbitrary",),
        ),
    )(x, w)


def _merge_desc(ka, va, kb, vb):
    kbr = lax.rev(kb, (0,))
    vbr = lax.rev(vb, (0,))
    take_a = ka >= kbr
    k = jnp.where(take_a, ka, kbr)
    v = jnp.where(take_a, va, vbr)
    return plsc.sort_key_val(k, v, descending=True)


def _sc_route(logits):
    rows = logits.shape[0]
    info = plsc.get_sparse_core_info()
    n_cores, n_subcores = info.num_cores, info.num_subcores
    n_workers = n_cores * n_subcores
    rpw = rows // n_workers
    mesh = plsc.VectorSubcoreMesh(core_axis_name="c", subcore_axis_name="s")

    @functools.partial(
        pl.kernel,
        out_type=(
            jax.ShapeDtypeStruct((rows * _TOP_K,), jnp.float32),
            jax.ShapeDtypeStruct((rows * _TOP_K,), jnp.int32),
        ),
        mesh=mesh,
        scratch_types=[
            pltpu.VMEM((rpw, _N_EXPERTS), jnp.float32),
            pltpu.VMEM((rpw * _TOP_K + 2 * _LANES,), jnp.float32),
            pltpu.VMEM((rpw * _TOP_K + 2 * _LANES,), jnp.int32),
        ],
        compiler_params=pltpu.CompilerParams(needs_layout_passes=False),
    )
    def route(lg_hbm, tv_hbm, ti_hbm, lg_v, tv_v, ti_v):
        wid = lax.axis_index("s") * n_cores + lax.axis_index("c")
        base = wid * rpw
        pltpu.sync_copy(lg_hbm.at[pl.ds(base, rpw)], lg_v)
        lane = lax.iota(jnp.int32, _LANES)
        top_mask = lane < _TOP_K

        def body(r, carry):
            lg = [lg_v[r, pl.ds(c * _LANES, _LANES)] for c in range(4)]
            m = jnp.max(
                jnp.maximum(jnp.maximum(lg[0], lg[1]), jnp.maximum(lg[2], lg[3]))
            )
            e = [jnp.exp(l - m) for l in lg]
            s = jnp.sum((e[0] + e[1]) + (e[2] + e[3]))
            ks, vs = [], []
            for c in range(4):
                p = e[c] / s
                idx = lane + c * _LANES
                key = jnp.where(p > 0, plsc.bitcast(p, jnp.int32), -idx)
                kk, vv = plsc.sort_key_val(key, idx, descending=True)
                ks.append(kk)
                vs.append(vv)
            k01, v01 = _merge_desc(ks[0], vs[0], ks[1], vs[1])
            k23, v23 = _merge_desc(ks[2], vs[2], ks[3], vs[3])
            kf, vf = _merge_desc(k01, v01, k23, v23)
            tv = jnp.where(kf > 0, plsc.bitcast(kf, jnp.float32), 0.0)
            tv = jnp.where(top_mask, tv, 0.0)
            tv = tv / jnp.sum(tv)
            plsc.store_compressed(
                tv_v.at[pl.ds(r * _TOP_K, _LANES)], tv, mask=top_mask
            )
            plsc.store_compressed(
                ti_v.at[pl.ds(r * _TOP_K, _LANES)], vf, mask=top_mask
            )
            return carry

        lax.fori_loop(0, rpw, body, 0, unroll=4)
        pltpu.sync_copy(
            tv_v.at[pl.ds(0, rpw * _TOP_K)],
            tv_hbm.at[pl.ds(base * _TOP_K, rpw * _TOP_K)],
        )
        pltpu.sync_copy(
            ti_v.at[pl.ds(0, rpw * _TOP_K)],
            ti_hbm.at[pl.ds(base * _TOP_K, rpw * _TOP_K)],
        )

    return route(logits)


def kernel(hidden_states, weight):
    x = hidden_states.reshape(-1, _HIDDEN)
    n = x.shape[0]
    chunk = n // _CHUNKS
    lg_parts, tv_parts, ti_parts = [], [], []
    for i in range(_CHUNKS):
        lg = _tc_logits(lax.slice(x, (i * chunk, 0), ((i + 1) * chunk, _HIDDEN)), weight)
        tv_flat, ti_flat = _sc_route(lg)
        lg_parts.append(lg)
        tv_parts.append(tv_flat.reshape(chunk, _TOP_K))
        ti_parts.append(ti_flat.reshape(chunk, _TOP_K))
    if _CHUNKS == 1:
        return lg_parts[0], tv_parts[0], ti_parts[0]
    return (
        jnp.concatenate(lg_parts, 0),
        jnp.concatenate(tv_parts, 0),
        jnp.concatenate(ti_parts, 0),
    )


# speedup vs baseline: 1.9569x; 1.0984x over previous
import functools

import jax
import jax.numpy as jnp
from jax import lax
from jax.experimental import pallas as pl
from jax.experimental.pallas import tpu as pltpu
from jax.experimental.pallas import tpu_sc as plsc

_HIDDEN = 4096
_N_EXPERTS = 64
_TOP_K = 8
_BLOCK = 1024
_CHUNKS = 1
_LANES = 16


def _matmul_body(x_ref, w_ref, logits_ref, keys_ref):
    logits = lax.dot_general(
        x_ref[...], w_ref[...], (((1,), (1,)), ((), ())),
        preferred_element_type=jnp.float32,
    )
    logits_ref[...] = logits
    m = jnp.max(logits, axis=-1, keepdims=True)
    e = jnp.exp(logits - m)
    p = e / jnp.sum(e, axis=-1, keepdims=True)
    col = lax.broadcasted_iota(jnp.int32, logits.shape, 1)
    keys_ref[...] = jnp.where(
        p > 0, lax.bitcast_convert_type(p, jnp.int32), -col
    )


def _tc_logits(x, w):
    n = x.shape[0]
    return pl.pallas_call(
        _matmul_body,
        grid=(n // _BLOCK,),
        in_specs=[
            pl.BlockSpec((_BLOCK, _HIDDEN), lambda i: (i, 0)),
            pl.BlockSpec((_N_EXPERTS, _HIDDEN), lambda i: (0, 0)),
        ],
        out_specs=(
            pl.BlockSpec((_BLOCK, _N_EXPERTS), lambda i: (i, 0)),
            pl.BlockSpec((_BLOCK, _N_EXPERTS), lambda i: (i, 0)),
        ),
        out_shape=(
            jax.ShapeDtypeStruct((n, _N_EXPERTS), jnp.float32),
            jax.ShapeDtypeStruct((n, _N_EXPERTS), jnp.int32),
        ),
        compiler_params=pltpu.CompilerParams(
            dimension_semantics=("arbitrary",),
        ),
    )(x, w)


def _merge_desc(ka, va, kb, vb):
    kbr = lax.rev(kb, (0,))
    vbr = lax.rev(vb, (0,))
    take_a = ka >= kbr
    k = jnp.where(take_a, ka, kbr)
    v = jnp.where(take_a, va, vbr)
    return plsc.sort_key_val(k, v, descending=True)


def _sc_route(keys):
    rows = keys.shape[0]
    info = plsc.get_sparse_core_info()
    n_cores, n_subcores = info.num_cores, info.num_subcores
    n_workers = n_cores * n_subcores
    rpw = rows // n_workers
    mesh = plsc.VectorSubcoreMesh(core_axis_name="c", subcore_axis_name="s")

    @functools.partial(
        pl.kernel,
        out_type=(
            jax.ShapeDtypeStruct((rows * _TOP_K,), jnp.float32),
            jax.ShapeDtypeStruct((rows * _TOP_K,), jnp.int32),
        ),
        mesh=mesh,
        scratch_types=[
            pltpu.VMEM((rpw, _N_EXPERTS), jnp.int32),
            pltpu.VMEM((rpw * _TOP_K + 2 * _LANES,), jnp.float32),
            pltpu.VMEM((rpw * _TOP_K + 2 * _LANES,), jnp.int32),
        ],
        compiler_params=pltpu.CompilerParams(needs_layout_passes=False),
    )
    def route(key_hbm, tv_hbm, ti_hbm, key_v, tv_v, ti_v):
        wid = lax.axis_index("s") * n_cores + lax.axis_index("c")
        base = wid * rpw
        pltpu.sync_copy(key_hbm.at[pl.ds(base, rpw)], key_v)
        lane = lax.iota(jnp.int32, _LANES)
        top_mask = lane < _TOP_K

        def body(r, carry):
            ks, vs = [], []
            for c in range(4):
                kk, vv = plsc.sort_key_val(
                    key_v[r, pl.ds(c * _LANES, _LANES)],
                    lane + c * _LANES,
                    descending=True,
                )
                ks.append(kk)
                vs.append(vv)
            k01, v01 = _merge_desc(ks[0], vs[0], ks[1], vs[1])
            k23, v23 = _merge_desc(ks[2], vs[2], ks[3], vs[3])
            kf, vf = _merge_desc(k01, v01, k23, v23)
            tv = jnp.where(kf > 0, plsc.bitcast(kf, jnp.float32), 0.0)
            tv = jnp.where(top_mask, tv, 0.0)
            tv = tv / jnp.sum(tv)
            plsc.store_compressed(
                tv_v.at[pl.ds(r * _TOP_K, _LANES)], tv, mask=top_mask
            )
            plsc.store_compressed(
                ti_v.at[pl.ds(r * _TOP_K, _LANES)], vf, mask=top_mask
            )
            return carry

        lax.fori_loop(0, rpw, body, 0, unroll=4)
        pltpu.sync_copy(
            tv_v.at[pl.ds(0, rpw * _TOP_K)],
            tv_hbm.at[pl.ds(base * _TOP_K, rpw * _TOP_K)],
        )
        pltpu.sync_copy(
            ti_v.at[pl.ds(0, rpw * _TOP_K)],
            ti_hbm.at[pl.ds(base * _TOP_K, rpw * _TOP_K)],
        )

    return route(keys)


def kernel(hidden_states, weight):
    x = hidden_states.reshape(-1, _HIDDEN)
    n = x.shape[0]
    chunk = n // _CHUNKS
    lg_parts, tv_parts, ti_parts = [], [], []
    for i in range(_CHUNKS):
        lg, keys = _tc_logits(
            lax.slice(x, (i * chunk, 0), ((i + 1) * chunk, _HIDDEN)), weight
        )
        tv_flat, ti_flat = _sc_route(keys)
        lg_parts.append(lg)
        tv_parts.append(tv_flat.reshape(chunk, _TOP_K))
        ti_parts.append(ti_flat.reshape(chunk, _TOP_K))
    if _CHUNKS == 1:
        return lg_parts[0], tv_parts[0], ti_parts[0]
    return (
        jnp.concatenate(lg_parts, 0),
        jnp.concatenate(tv_parts, 0),
        jnp.concatenate(ti_parts, 0),
    )


# speedup vs baseline: 2.2958x; 1.1732x over previous
import functools

import jax
import jax.numpy as jnp
from jax import lax
from jax.experimental import pallas as pl
from jax.experimental.pallas import tpu as pltpu
from jax.experimental.pallas import tpu_sc as plsc

_HIDDEN = 4096
_N_EXPERTS = 64
_TOP_K = 8
_BLOCK = 1024
_CHUNKS = 1
_LANES = 16


def _matmul_body(x_ref, w_ref, logits_ref, keys_ref):
    logits = lax.dot_general(
        x_ref[...], w_ref[...], (((1,), (1,)), ((), ())),
        preferred_element_type=jnp.float32,
    )
    logits_ref[...] = logits
    m = jnp.max(logits, axis=-1, keepdims=True)
    e = jnp.exp(logits - m)
    p = e / jnp.sum(e, axis=-1, keepdims=True)
    col = lax.broadcasted_iota(jnp.int32, logits.shape, 1)
    keys_ref[...] = jnp.where(
        p > 0, lax.bitcast_convert_type(p, jnp.int32), -col
    )


def _tc_logits(x, w):
    n = x.shape[0]
    return pl.pallas_call(
        _matmul_body,
        grid=(n // _BLOCK,),
        in_specs=[
            pl.BlockSpec((_BLOCK, _HIDDEN), lambda i: (i, 0)),
            pl.BlockSpec((_N_EXPERTS, _HIDDEN), lambda i: (0, 0)),
        ],
        out_specs=(
            pl.BlockSpec((_BLOCK, _N_EXPERTS), lambda i: (i, 0)),
            pl.BlockSpec((_BLOCK, _N_EXPERTS), lambda i: (i, 0)),
        ),
        out_shape=(
            jax.ShapeDtypeStruct((n, _N_EXPERTS), jnp.float32),
            jax.ShapeDtypeStruct((n, _N_EXPERTS), jnp.int32),
        ),
        compiler_params=pltpu.CompilerParams(
            dimension_semantics=("arbitrary",),
        ),
    )(x, w)


def _take16(x, idx):
    return lax.gather(
        x,
        idx[:, None],
        lax.GatherDimensionNumbers(
            offset_dims=(), collapsed_slice_dims=(0,), start_index_map=(0,)
        ),
        slice_sizes=(1,),
        mode=lax.GatherScatterMode.PROMISE_IN_BOUNDS,
    )


def _merge_desc(ka, va, kb, vb):
    kbr = lax.rev(kb, (0,))
    vbr = lax.rev(vb, (0,))
    take_a = ka >= kbr
    k = jnp.where(take_a, ka, kbr)
    v = jnp.where(take_a, va, vbr)
    return plsc.sort_key_val(k, v, descending=True)


def _sc_route(keys):
    rows = keys.shape[0]
    info = plsc.get_sparse_core_info()
    n_cores, n_subcores = info.num_cores, info.num_subcores
    n_workers = n_cores * n_subcores
    rpw = rows // n_workers
    mesh = plsc.VectorSubcoreMesh(core_axis_name="c", subcore_axis_name="s")

    @functools.partial(
        pl.kernel,
        out_type=(
            jax.ShapeDtypeStruct((rows * _TOP_K,), jnp.float32),
            jax.ShapeDtypeStruct((rows * _TOP_K,), jnp.int32),
        ),
        mesh=mesh,
        scratch_types=[
            pltpu.VMEM((rpw, _N_EXPERTS), jnp.int32),
            pltpu.VMEM((rpw * _TOP_K + 2 * _LANES,), jnp.float32),
            pltpu.VMEM((rpw * _TOP_K + 2 * _LANES,), jnp.int32),
        ],
        compiler_params=pltpu.CompilerParams(needs_layout_passes=False),
    )
    def route(key_hbm, tv_hbm, ti_hbm, key_v, tv_v, ti_v):
        wid = lax.axis_index("s") * n_cores + lax.axis_index("c")
        base = wid * rpw
        pltpu.sync_copy(key_hbm.at[pl.ds(base, rpw)], key_v)
        lane = lax.iota(jnp.int32, _LANES)
        top_mask = lane < _TOP_K
        shift_idx = jnp.where(top_mask, 0, lane - _TOP_K)

        def top8_row(r):
            ks, vs = [], []
            for c in range(4):
                kk, vv = plsc.sort_key_val(
                    key_v[r, pl.ds(c * _LANES, _LANES)],
                    lane + c * _LANES,
                    descending=True,
                )
                ks.append(kk)
                vs.append(vv)
            k01, v01 = _merge_desc(ks[0], vs[0], ks[1], vs[1])
            k23, v23 = _merge_desc(ks[2], vs[2], ks[3], vs[3])
            kf, vf = _merge_desc(k01, v01, k23, v23)
            tv = jnp.where(kf > 0, plsc.bitcast(kf, jnp.float32), 0.0)
            tv = jnp.where(top_mask, tv, 0.0)
            return tv / jnp.sum(tv), vf

        @plsc.parallel_loop(0, rpw, step=2, unroll=4)
        def body(r):
            tv_a, ti_a = top8_row(r)
            tv_b, ti_b = top8_row(r + 1)
            tv_b = _take16(tv_b, shift_idx)
            ti_b = _take16(ti_b, shift_idx)
            tv_v[pl.ds(r * _TOP_K, _LANES)] = jnp.where(top_mask, tv_a, tv_b)
            ti_v[pl.ds(r * _TOP_K, _LANES)] = jnp.where(top_mask, ti_a, ti_b)
        pltpu.sync_copy(
            tv_v.at[pl.ds(0, rpw * _TOP_K)],
            tv_hbm.at[pl.ds(base * _TOP_K, rpw * _TOP_K)],
        )
        pltpu.sync_copy(
            ti_v.at[pl.ds(0, rpw * _TOP_K)],
            ti_hbm.at[pl.ds(base * _TOP_K, rpw * _TOP_K)],
        )

    return route(keys)


def kernel(hidden_states, weight):
    x = hidden_states.reshape(-1, _HIDDEN)
    n = x.shape[0]
    chunk = n // _CHUNKS
    lg_parts, tv_parts, ti_parts = [], [], []
    for i in range(_CHUNKS):
        lg, keys = _tc_logits(
            lax.slice(x, (i * chunk, 0), ((i + 1) * chunk, _HIDDEN)), weight
        )
        tv_flat, ti_flat = _sc_route(keys)
        lg_parts.append(lg)
        tv_parts.append(tv_flat.reshape(chunk, _TOP_K))
        ti_parts.append(ti_flat.reshape(chunk, _TOP_K))
    if _CHUNKS == 1:
        return lg_parts[0], tv_parts[0], ti_parts[0]
    return (
        jnp.concatenate(lg_parts, 0),
        jnp.concatenate(tv_parts, 0),
        jnp.concatenate(ti_parts, 0),
    )
